# baseline (speedup 1.0000x reference)
import functools

import jax
import jax.numpy as jnp
from jax import lax
from jax.experimental import pallas as pl
from jax.experimental.pallas import tpu as pltpu
from jax.experimental.pallas import tpu_sc as plsc

N_LIG = 2048
N_PROT = 16384
N_PROT_PAD = N_PROT + 16
N_BATCH = 32
D2_MAX = 4.5 * 4.5
K_NBR = 15
EPS = 1e-8

NC = 1
NW = NC * 16
NLAUNCH = 2
QPW = N_LIG // (NW * NLAUNCH)
L = 16
QN = 4

_INF = float("inf")


def _sqrt16(x):
    xi = plsc.bitcast(x, jnp.int32)
    y = plsc.bitcast(jnp.int32(0x5F3759DF) - (xi >> 1), jnp.float32)
    for _ in range(3):
        y = y * (1.5 - 0.5 * x * y * y)
    return x * y


def _sc_body(qoff, lgx_h, lgy_h, lgz_h, ltx_h, lty_h, ltz_h,
             pgx_h, pgy_h, pgz_h, ptx_h, pty_h, ptz_h,
             lb_hbm, pb_hbm, tw_hbm,
             se_hbm, cnt_hbm,
             lgx, lgy, lgz, ltx, lty, ltz, lb, tww, cnts,
             pgx, pgy, pgz, ptx, pty, ptz, pb, ovec, dsem):
    wid = lax.axis_index("s") * NC + lax.axis_index("c")
    base = qoff + wid * QPW

    copies = [
        pltpu.async_copy(pgx_h, pgx, dsem),
        pltpu.async_copy(pgy_h, pgy, dsem),
        pltpu.async_copy(pgz_h, pgz, dsem),
        pltpu.async_copy(ptx_h, ptx, dsem),
        pltpu.async_copy(pty_h, pty, dsem),
        pltpu.async_copy(ptz_h, ptz, dsem),
        pltpu.async_copy(pb_hbm, pb, dsem),
        pltpu.async_copy(lgx_h.at[pl.ds(base, QPW)], lgx, dsem),
        pltpu.async_copy(lgy_h.at[pl.ds(base, QPW)], lgy, dsem),
        pltpu.async_copy(lgz_h.at[pl.ds(base, QPW)], lgz, dsem),
        pltpu.async_copy(ltx_h.at[pl.ds(base, QPW)], ltx, dsem),
        pltpu.async_copy(lty_h.at[pl.ds(base, QPW)], lty, dsem),
        pltpu.async_copy(ltz_h.at[pl.ds(base, QPW)], ltz, dsem),
        pltpu.async_copy(lb_hbm.at[pl.ds(base, QPW)], lb, dsem),
        pltpu.async_copy(tw_hbm, tww, dsem),
    ]
    for c in copies:
        c.wait()

    lane = lax.iota(jnp.int32, L)

    for g in range(3):
        bvec = lane + g * L
        lo = jnp.zeros((L,), jnp.int32)
        p = N_PROT // 2
        while p >= 1:
            v = plsc.load_gather(pb, [lo + (p - 1)])
            lo = jnp.where(v < bvec, lo + p, lo)
            p //= 2
        v = plsc.load_gather(pb, [lo])
        lo = jnp.where(v < bvec, lo + 1, lo)
        cnts[pl.ds(g * L, L)] = lo

    inf16 = jnp.full((L,), _INF)

    def _finish(av, ap, isplat, twv, ca, na):
        validm = (av <= D2_MAX) & (lane < K_NBR)
        d2t = jnp.minimum(av, 1e8)
        gx = plsc.load_gather(lgx, [isplat]) - plsc.load_gather(pgx, [ap])
        gy = plsc.load_gather(lgy, [isplat]) - plsc.load_gather(pgy, [ap])
        gz = plsc.load_gather(lgz, [isplat]) - plsc.load_gather(pgz, [ap])
        d2g = gx * gx + gy * gy + gz * gz
        dij_g = _sqrt16(jnp.maximum(d2g, EPS))
        dij_t = _sqrt16(jnp.maximum(d2t, EPS))
        se = (dij_g - dij_t) * (dij_g - dij_t)
        ca = ca + jnp.where(validm, se * twv, 0.0)
        na = na + jnp.where(validm, 1.0, 0.0)
        return ca, na

    zeroi = jnp.zeros((L,), jnp.int32)

    def qbody(q, carry):
        ca, na = carry
        iq = [jnp.full((L,), QN * q + k, jnp.int32) for k in range(QN)]
        bq = [plsc.load_gather(lb, [i]) for i in iq]
        sq = [plsc.load_gather(cnts, [b]) for b in bq]
        eq = [plsc.load_gather(cnts, [b + 1]) for b in bq]
        twq = [plsc.load_gather(tww, [b]) for b in bq]
        ltxq = [plsc.load_gather(ltx, [i]) for i in iq]
        ltyq = [plsc.load_gather(lty, [i]) for i in iq]
        ltzq = [plsc.load_gather(ltz, [i]) for i in iq]
        s0 = sq[0][0] & ~(L - 1)
        nblk = (eq[-1][0] - s0 + (L - 1)) >> 4

        def tblock(t, c2):
            avs, aps = list(c2[:QN]), list(c2[QN:])
            j0 = s0 + t * L
            posv = lane + j0
            px = ptx[pl.ds(j0, L)]
            py = pty[pl.ds(j0, L)]
            pz = ptz[pl.ds(j0, L)]
            for k in range(QN):
                dx = px - ltxq[k]
                dy = py - ltyq[k]
                dz = pz - ltzq[k]
                d2 = dx * dx + dy * dy + dz * dz
                d2 = jnp.where((posv >= sq[k]) & (posv < eq[k]), d2, _INF)
                bv, bp = plsc.sort_key_val(d2, posv, descending=True)
                tk = bv < avs[k]
                mv = jnp.where(tk, bv, avs[k])
                mp = jnp.where(tk, bp, aps[k])
                avs[k], aps[k] = plsc.sort_key_val(mv, mp)
            return tuple(avs) + tuple(aps)

        res = lax.fori_loop(
            0, nblk, tblock, (inf16,) * QN + (zeroi,) * QN)

        for k in range(QN):
            ca, na = _finish(res[k], res[QN + k], iq[k], twq[k], ca, na)
        return ca, na

    zero16 = jnp.zeros((L,), jnp.float32)
    ca, na = lax.fori_loop(0, QPW // QN, qbody, (zero16, zero16))
    ovec[...] = ca
    pltpu.sync_copy(ovec, se_hbm.at[wid])
    ovec[...] = na
    pltpu.sync_copy(ovec, cnt_hbm.at[wid])


@jax.jit
def _run(lgx, lgy, lgz, ltx, lty, ltz, pgx, pgy, pgz, ptx, pty, ptz,
         lb, pb, tw):
    mesh = plsc.VectorSubcoreMesh(core_axis_name="c", subcore_axis_name="s",
                                  num_cores=NC, num_subcores=16)
    f32, i32 = jnp.float32, jnp.int32

    def make(qoff):
        return pl.kernel(
            functools.partial(_sc_body, qoff),
            out_type=(
                jax.ShapeDtypeStruct((NW, L), f32),
                jax.ShapeDtypeStruct((NW, L), f32),
            ),
            mesh=mesh,
            compiler_params=pltpu.CompilerParams(needs_layout_passes=False),
            scratch_types=(
                pltpu.VMEM((QPW,), f32), pltpu.VMEM((QPW,), f32),
                pltpu.VMEM((QPW,), f32), pltpu.VMEM((QPW,), f32),
                pltpu.VMEM((QPW,), f32), pltpu.VMEM((QPW,), f32),
                pltpu.VMEM((QPW,), i32),
                pltpu.VMEM((N_BATCH,), f32),
                pltpu.VMEM((3 * L,), i32),
                pltpu.VMEM((N_PROT_PAD,), f32), pltpu.VMEM((N_PROT_PAD,), f32),
                pltpu.VMEM((N_PROT_PAD,), f32),
                pltpu.VMEM((N_PROT_PAD,), f32), pltpu.VMEM((N_PROT_PAD,), f32),
                pltpu.VMEM((N_PROT_PAD,), f32),
                pltpu.VMEM((N_PROT,), i32),
                pltpu.VMEM((L,), f32),
                pltpu.SemaphoreType.DMA,
            ),
        )

    args = (lgx, lgy, lgz, ltx, lty, ltz, pgx, pgy, pgz, ptx, pty, ptz,
            lb, pb, tw)
    se0, cnt0 = make(0)(*args)
    se1, cnt1 = make(NW * QPW)(*args)
    total = jnp.sum(se0) + jnp.sum(se1)
    n = jnp.sum(cnt0) + jnp.sum(cnt1)
    return total / jnp.maximum(n, 1.0)


def kernel(lig_x_gen, prot_x_gen, lig_x_true, prot_x_true, lig_batch,
           prot_batch, time_weights):
    pad = jnp.zeros((N_PROT_PAD - N_PROT,), jnp.float32)
    pg = [jnp.concatenate([prot_x_gen[:, c], pad]) for c in range(3)]
    pt = [jnp.concatenate([prot_x_true[:, c], pad]) for c in range(3)]
    lg = [lig_x_gen[:, c] for c in range(3)]
    lt = [lig_x_true[:, c] for c in range(3)]
    return _run(
        *lg, *lt, *pg, *pt,
        lig_batch.astype(jnp.int32), prot_batch.astype(jnp.int32),
        time_weights,
    )


# speedup vs baseline: 1.2539x; 1.2539x over previous
import functools

import jax
import jax.numpy as jnp
from jax import lax
from jax.experimental import pallas as pl
from jax.experimental.pallas import tpu as pltpu
from jax.experimental.pallas import tpu_sc as plsc

N_LIG = 2048
N_PROT = 16384
N_PROT_PAD = N_PROT + 16
N_BATCH = 32
D2_MAX = 4.5 * 4.5
K_NBR = 15
EPS = 1e-8

NC = 2
NW = NC * 16
QPW = N_LIG // NW
L = 16
QN = 4

_INF = float("inf")


def _sqrt16(x):
    xi = plsc.bitcast(x, jnp.int32)
    y = plsc.bitcast(jnp.int32(0x5F3759DF) - (xi >> 1), jnp.float32)
    for _ in range(3):
        y = y * (1.5 - 0.5 * x * y * y)
    return x * y


def _sc_body(qoff, lgx_h, lgy_h, lgz_h, ltx_h, lty_h, ltz_h,
             pgx_h, pgy_h, pgz_h, ptx_h, pty_h, ptz_h,
             lb_hbm, pb_hbm, tw_hbm,
             se_hbm, cnt_hbm,
             lgx, lgy, lgz, ltx, lty, ltz, lb, tww, cnts,
             pgx, pgy, pgz, ptx, pty, ptz, pb, ovec, dsem):
    wid = lax.axis_index("s") * NC + lax.axis_index("c")
    base = qoff + wid * QPW

    copies = [
        pltpu.async_copy(pgx_h, pgx, dsem),
        pltpu.async_copy(pgy_h, pgy, dsem),
        pltpu.async_copy(pgz_h, pgz, dsem),
        pltpu.async_copy(ptx_h, ptx, dsem),
        pltpu.async_copy(pty_h, pty, dsem),
        pltpu.async_copy(ptz_h, ptz, dsem),
        pltpu.async_copy(pb_hbm, pb, dsem),
        pltpu.async_copy(lgx_h.at[pl.ds(base, QPW)], lgx, dsem),
        pltpu.async_copy(lgy_h.at[pl.ds(base, QPW)], lgy, dsem),
        pltpu.async_copy(lgz_h.at[pl.ds(base, QPW)], lgz, dsem),
        pltpu.async_copy(ltx_h.at[pl.ds(base, QPW)], ltx, dsem),
        pltpu.async_copy(lty_h.at[pl.ds(base, QPW)], lty, dsem),
        pltpu.async_copy(ltz_h.at[pl.ds(base, QPW)], ltz, dsem),
        pltpu.async_copy(lb_hbm.at[pl.ds(base, QPW)], lb, dsem),
        pltpu.async_copy(tw_hbm, tww, dsem),
    ]
    for c in copies:
        c.wait()

    lane = lax.iota(jnp.int32, L)

    for g in range(3):
        bvec = lane + g * L
        lo = jnp.zeros((L,), jnp.int32)
        p = N_PROT // 2
        while p >= 1:
            v = plsc.load_gather(pb, [lo + (p - 1)])
            lo = jnp.where(v < bvec, lo + p, lo)
            p //= 2
        v = plsc.load_gather(pb, [lo])
        lo = jnp.where(v < bvec, lo + 1, lo)
        cnts[pl.ds(g * L, L)] = lo

    inf16 = jnp.full((L,), _INF)

    def _finish(av, ap, isplat, twv, ca, na):
        validm = (av <= D2_MAX) & (lane < K_NBR)
        d2t = jnp.minimum(av, 1e8)
        gx = plsc.load_gather(lgx, [isplat]) - plsc.load_gather(pgx, [ap])
        gy = plsc.load_gather(lgy, [isplat]) - plsc.load_gather(pgy, [ap])
        gz = plsc.load_gather(lgz, [isplat]) - plsc.load_gather(pgz, [ap])
        d2g = gx * gx + gy * gy + gz * gz
        dij_g = _sqrt16(jnp.maximum(d2g, EPS))
        dij_t = _sqrt16(jnp.maximum(d2t, EPS))
        se = (dij_g - dij_t) * (dij_g - dij_t)
        ca = ca + jnp.where(validm, se * twv, 0.0)
        na = na + jnp.where(validm, 1.0, 0.0)
        return ca, na

    zeroi = jnp.zeros((L,), jnp.int32)

    def qbody(q, carry):
        ca, na = carry
        iq = [jnp.full((L,), QN * q + k, jnp.int32) for k in range(QN)]
        bq = [plsc.load_gather(lb, [i]) for i in iq]
        sq = [plsc.load_gather(cnts, [b]) for b in bq]
        eq = [plsc.load_gather(cnts, [b + 1]) for b in bq]
        twq = [plsc.load_gather(tww, [b]) for b in bq]
        ltxq = [plsc.load_gather(ltx, [i]) for i in iq]
        ltyq = [plsc.load_gather(lty, [i]) for i in iq]
        ltzq = [plsc.load_gather(ltz, [i]) for i in iq]
        s0 = sq[0][0] & ~(L - 1)
        nblk = (eq[-1][0] - s0 + (L - 1)) >> 4

        def tblock(t, c2):
            avs, aps = list(c2[:QN]), list(c2[QN:])
            j0 = s0 + t * L
            posv = lane + j0
            px = ptx[pl.ds(j0, L)]
            py = pty[pl.ds(j0, L)]
            pz = ptz[pl.ds(j0, L)]
            for k in range(QN):
                dx = px - ltxq[k]
                dy = py - ltyq[k]
                dz = pz - ltzq[k]
                d2 = dx * dx + dy * dy + dz * dz
                d2 = jnp.where((posv >= sq[k]) & (posv < eq[k]), d2, _INF)
                bv, bp = plsc.sort_key_val(d2, posv, descending=True)
                tk = bv < avs[k]
                mv = jnp.where(tk, bv, avs[k])
                mp = jnp.where(tk, bp, aps[k])
                avs[k], aps[k] = plsc.sort_key_val(mv, mp)
            return tuple(avs) + tuple(aps)

        res = lax.fori_loop(
            0, nblk, tblock, (inf16,) * QN + (zeroi,) * QN)

        for k in range(QN):
            ca, na = _finish(res[k], res[QN + k], iq[k], twq[k], ca, na)
        return ca, na

    zero16 = jnp.zeros((L,), jnp.float32)
    ca, na = lax.fori_loop(0, QPW // QN, qbody, (zero16, zero16))
    ovec[...] = ca
    pltpu.sync_copy(ovec, se_hbm.at[wid])
    ovec[...] = na
    pltpu.sync_copy(ovec, cnt_hbm.at[wid])


@jax.jit
def _run(lgx, lgy, lgz, ltx, lty, ltz, pgx, pgy, pgz, ptx, pty, ptz,
         lb, pb, tw):
    mesh = plsc.VectorSubcoreMesh(core_axis_name="c", subcore_axis_name="s",
                                  num_cores=NC, num_subcores=16)
    f32, i32 = jnp.float32, jnp.int32

    call = pl.kernel(
            functools.partial(_sc_body, 0),
            out_type=(
                jax.ShapeDtypeStruct((NW, L), f32),
                jax.ShapeDtypeStruct((NW, L), f32),
            ),
            mesh=mesh,
            compiler_params=pltpu.CompilerParams(needs_layout_passes=False),
            scratch_types=(
                pltpu.VMEM((QPW,), f32), pltpu.VMEM((QPW,), f32),
                pltpu.VMEM((QPW,), f32), pltpu.VMEM((QPW,), f32),
                pltpu.VMEM((QPW,), f32), pltpu.VMEM((QPW,), f32),
                pltpu.VMEM((QPW,), i32),
                pltpu.VMEM((N_BATCH,), f32),
                pltpu.VMEM((3 * L,), i32),
                pltpu.VMEM((N_PROT_PAD,), f32), pltpu.VMEM((N_PROT_PAD,), f32),
                pltpu.VMEM((N_PROT_PAD,), f32),
                pltpu.VMEM((N_PROT_PAD,), f32), pltpu.VMEM((N_PROT_PAD,), f32),
                pltpu.VMEM((N_PROT_PAD,), f32),
                pltpu.VMEM((N_PROT,), i32),
                pltpu.VMEM((L,), f32),
                pltpu.SemaphoreType.DMA,
            ),
        )

    se0, cnt0 = call(lgx, lgy, lgz, ltx, lty, ltz, pgx, pgy, pgz,
                     ptx, pty, ptz, lb, pb, tw)
    return jnp.sum(se0) / jnp.maximum(jnp.sum(cnt0), 1.0)


def kernel(lig_x_gen, prot_x_gen, lig_x_true, prot_x_true, lig_batch,
           prot_batch, time_weights):
    pad = jnp.zeros((N_PROT_PAD - N_PROT,), jnp.float32)
    pg = [jnp.concatenate([prot_x_gen[:, c], pad]) for c in range(3)]
    pt = [jnp.concatenate([prot_x_true[:, c], pad]) for c in range(3)]
    lg = [lig_x_gen[:, c] for c in range(3)]
    lt = [lig_x_true[:, c] for c in range(3)]
    return _run(
        *lg, *lt, *pg, *pt,
        lig_batch.astype(jnp.int32), prot_batch.astype(jnp.int32),
        time_weights,
    )


# speedup vs baseline: 1.3536x; 1.0795x over previous
import functools

import jax
import jax.numpy as jnp
from jax import lax
from jax.experimental import pallas as pl
from jax.experimental.pallas import tpu as pltpu
from jax.experimental.pallas import tpu_sc as plsc

N_LIG = 2048
N_PROT = 16384
N_BATCH = 32
D2_MAX = 4.5 * 4.5
K_NBR = 15
EPS = 1e-8

NC = 2
NW = NC * 16
QPW = N_LIG // NW
L = 16
QN = 4

_INF = float("inf")


def _sqrt16(x):
    xi = plsc.bitcast(x, jnp.int32)
    y = plsc.bitcast(jnp.int32(0x5F3759DF) - (xi >> 1), jnp.float32)
    for _ in range(3):
        y = y * (1.5 - 0.5 * x * y * y)
    return x * y


def _sc_body(qoff, lgx_h, lgy_h, lgz_h, ltx_h, lty_h, ltz_h,
             pgx_h, pgy_h, pgz_h, ptx_h, pty_h, ptz_h,
             lb_hbm, pb_hbm, tw_hbm,
             se_hbm, cnt_hbm,
             lgx, lgy, lgz, ltx, lty, ltz, lb, tww, cnts,
             pgx, pgy, pgz, ptx, pty, ptz, pb, ovec, dsem):
    wid = lax.axis_index("s") * NC + lax.axis_index("c")
    base = qoff + wid * QPW

    copies = [
        pltpu.async_copy(pgx_h, pgx, dsem),
        pltpu.async_copy(pgy_h, pgy, dsem),
        pltpu.async_copy(pgz_h, pgz, dsem),
        pltpu.async_copy(ptx_h, ptx, dsem),
        pltpu.async_copy(pty_h, pty, dsem),
        pltpu.async_copy(ptz_h, ptz, dsem),
        pltpu.async_copy(pb_hbm, pb, dsem),
        pltpu.async_copy(lgx_h.at[pl.ds(base, QPW)], lgx, dsem),
        pltpu.async_copy(lgy_h.at[pl.ds(base, QPW)], lgy, dsem),
        pltpu.async_copy(lgz_h.at[pl.ds(base, QPW)], lgz, dsem),
        pltpu.async_copy(ltx_h.at[pl.ds(base, QPW)], ltx, dsem),
        pltpu.async_copy(lty_h.at[pl.ds(base, QPW)], lty, dsem),
        pltpu.async_copy(ltz_h.at[pl.ds(base, QPW)], ltz, dsem),
        pltpu.async_copy(lb_hbm.at[pl.ds(base, QPW)], lb, dsem),
        pltpu.async_copy(tw_hbm, tww, dsem),
    ]
    for c in copies:
        c.wait()

    lane = lax.iota(jnp.int32, L)

    for g in range(3):
        bvec = lane + g * L
        lo = jnp.zeros((L,), jnp.int32)
        p = N_PROT // 2
        while p >= 1:
            v = plsc.load_gather(pb, [lo + (p - 1)])
            lo = jnp.where(v < bvec, lo + p, lo)
            p //= 2
        v = plsc.load_gather(pb, [lo])
        lo = jnp.where(v < bvec, lo + 1, lo)
        cnts[pl.ds(g * L, L)] = lo

    inf16 = jnp.full((L,), _INF)

    def _finish(av, ap, isplat, twv, ca, na):
        validm = (av <= D2_MAX) & (lane < K_NBR)
        d2t = jnp.minimum(av, 1e8)
        gx = plsc.load_gather(lgx, [isplat]) - plsc.load_gather(pgx, [ap])
        gy = plsc.load_gather(lgy, [isplat]) - plsc.load_gather(pgy, [ap])
        gz = plsc.load_gather(lgz, [isplat]) - plsc.load_gather(pgz, [ap])
        d2g = gx * gx + gy * gy + gz * gz
        dij_g = _sqrt16(jnp.maximum(d2g, EPS))
        dij_t = _sqrt16(jnp.maximum(d2t, EPS))
        se = (dij_g - dij_t) * (dij_g - dij_t)
        ca = ca + jnp.where(validm, se * twv, 0.0)
        na = na + jnp.where(validm, 1.0, 0.0)
        return ca, na

    zeroi = jnp.zeros((L,), jnp.int32)

    def qbody(q, carry):
        ca, na = carry
        iq = [jnp.full((L,), QN * q + k, jnp.int32) for k in range(QN)]
        bq = [plsc.load_gather(lb, [i]) for i in iq]
        sq = [plsc.load_gather(cnts, [b]) for b in bq]
        eq = [plsc.load_gather(cnts, [b + 1]) for b in bq]
        twq = [plsc.load_gather(tww, [b]) for b in bq]
        ltxq = [plsc.load_gather(ltx, [i]) for i in iq]
        ltyq = [plsc.load_gather(lty, [i]) for i in iq]
        ltzq = [plsc.load_gather(ltz, [i]) for i in iq]
        s0 = sq[0][0] & ~(L - 1)
        nblk = (eq[-1][0] - s0 + (L - 1)) >> 4

        def tblock(t, c2):
            avs, aps = list(c2[:QN]), list(c2[QN:])
            j0 = s0 + t * L
            posv = lane + j0
            px = ptx[pl.ds(j0, L)]
            py = pty[pl.ds(j0, L)]
            pz = ptz[pl.ds(j0, L)]
            for k in range(QN):
                dx = px - ltxq[k]
                dy = py - ltyq[k]
                dz = pz - ltzq[k]
                d2 = dx * dx + dy * dy + dz * dz
                d2 = jnp.where((posv >= sq[k]) & (posv < eq[k]), d2, _INF)
                bv, bp = plsc.sort_key_val(d2, posv, descending=True)
                tk = bv < avs[k]
                mv = jnp.where(tk, bv, avs[k])
                mp = jnp.where(tk, bp, aps[k])
                avs[k], aps[k] = plsc.sort_key_val(mv, mp)
            return tuple(avs) + tuple(aps)

        res = lax.fori_loop(
            0, nblk, tblock, (inf16,) * QN + (zeroi,) * QN)

        for k in range(QN):
            ca, na = _finish(res[k], res[QN + k], iq[k], twq[k], ca, na)
        return ca, na

    zero16 = jnp.zeros((L,), jnp.float32)
    ca, na = lax.fori_loop(0, QPW // QN, qbody, (zero16, zero16))
    ovec[...] = ca
    pltpu.sync_copy(ovec, se_hbm.at[wid])
    ovec[...] = na
    pltpu.sync_copy(ovec, cnt_hbm.at[wid])


@jax.jit
def _run(lgx, lgy, lgz, ltx, lty, ltz, pgx, pgy, pgz, ptx, pty, ptz,
         lb, pb, tw):
    mesh = plsc.VectorSubcoreMesh(core_axis_name="c", subcore_axis_name="s",
                                  num_cores=NC, num_subcores=16)
    f32, i32 = jnp.float32, jnp.int32

    call = pl.kernel(
            functools.partial(_sc_body, 0),
            out_type=(
                jax.ShapeDtypeStruct((NW, L), f32),
                jax.ShapeDtypeStruct((NW, L), f32),
            ),
            mesh=mesh,
            compiler_params=pltpu.CompilerParams(needs_layout_passes=False),
            scratch_types=(
                pltpu.VMEM((QPW,), f32), pltpu.VMEM((QPW,), f32),
                pltpu.VMEM((QPW,), f32), pltpu.VMEM((QPW,), f32),
                pltpu.VMEM((QPW,), f32), pltpu.VMEM((QPW,), f32),
                pltpu.VMEM((QPW,), i32),
                pltpu.VMEM((N_BATCH,), f32),
                pltpu.VMEM((3 * L,), i32),
                pltpu.VMEM((N_PROT,), f32), pltpu.VMEM((N_PROT,), f32),
                pltpu.VMEM((N_PROT,), f32),
                pltpu.VMEM((N_PROT,), f32), pltpu.VMEM((N_PROT,), f32),
                pltpu.VMEM((N_PROT,), f32),
                pltpu.VMEM((N_PROT,), i32),
                pltpu.VMEM((L,), f32),
                pltpu.SemaphoreType.DMA,
            ),
        )

    se0, cnt0 = call(lgx, lgy, lgz, ltx, lty, ltz, pgx, pgy, pgz,
                     ptx, pty, ptz, lb, pb, tw)
    return jnp.sum(se0) / jnp.maximum(jnp.sum(cnt0), 1.0)


def kernel(lig_x_gen, prot_x_gen, lig_x_true, prot_x_true, lig_batch,
           prot_batch, time_weights):
    pg = [prot_x_gen[:, c] for c in range(3)]
    pt = [prot_x_true[:, c] for c in range(3)]
    lg = [lig_x_gen[:, c] for c in range(3)]
    lt = [lig_x_true[:, c] for c in range(3)]
    return _run(
        *lg, *lt, *pg, *pt,
        lig_batch.astype(jnp.int32), prot_batch.astype(jnp.int32),
        time_weights,
    )
